# baseline (speedup 1.0000x reference)
import functools

import jax
import jax.numpy as jnp
from jax import lax
from jax.experimental import pallas as pl
from jax.experimental.pallas import tpu as pltpu
from jax.experimental.pallas import tpu_sc as plsc

_B = 16384
_D = 32

_cached = {}


def _make_kernel():
    if "k" in _cached:
        return _cached["k"]
    info = plsc.get_sparse_core_info()
    nc, ns = info.num_cores, info.num_subcores
    nw = nc * ns
    b_per_w = _B // nw
    mesh = plsc.VectorSubcoreMesh(core_axis_name="c", subcore_axis_name="s")

    @functools.partial(
        pl.kernel,
        mesh=mesh,
        out_type=jax.ShapeDtypeStruct((_B, _D), jnp.float32),
        scratch_types=[
            pltpu.VMEM((b_per_w,), jnp.int32),
            pltpu.VMEM((b_per_w, _D), jnp.float32),
            pltpu.SemaphoreType.DMA,
        ],
        compiler_params=pltpu.CompilerParams(use_tc_tiling_on_sc=False),
    )
    def k(t_hbm, table_hbm, out_hbm, idx_v, rows_v, sem):
        wid = lax.axis_index("s") * nc + lax.axis_index("c")
        base = wid * b_per_w
        pltpu.sync_copy(t_hbm.at[pl.ds(base, b_per_w)], idx_v)
        pltpu.async_copy(table_hbm.at[idx_v], rows_v, sem).wait()
        pltpu.sync_copy(rows_v, out_hbm.at[pl.ds(base, b_per_w)])

    _cached["k"] = k
    return k


def kernel(t, table):
    k = _make_kernel()
    return k(t.astype(jnp.int32), table.astype(jnp.float32))


# speedup vs baseline: 1.8258x; 1.8258x over previous
import functools

import jax
import jax.numpy as jnp
from jax import lax
from jax.experimental import pallas as pl
from jax.experimental.pallas import tpu as pltpu
from jax.experimental.pallas import tpu_sc as plsc

_B = 16384
_V = 10
_D = 32

_cached = {}


def _make_kernel():
    if "k" in _cached:
        return _cached["k"]
    info = plsc.get_sparse_core_info()
    nc, ns, nl = info.num_cores, info.num_subcores, info.num_lanes
    nw = nc * ns
    b_per_w = _B // nw
    n_chunks = b_per_w // nl
    mesh = plsc.VectorSubcoreMesh(core_axis_name="c", subcore_axis_name="s")

    @functools.partial(
        pl.kernel,
        mesh=mesh,
        out_type=jax.ShapeDtypeStruct((_B * _D,), jnp.float32),
        scratch_types=[
            pltpu.VMEM((b_per_w,), jnp.int32),
            pltpu.VMEM((_V * _D,), jnp.float32),
            pltpu.VMEM((b_per_w * _D,), jnp.float32),
        ],
        compiler_params=pltpu.CompilerParams(
            use_tc_tiling_on_sc=False, needs_layout_passes=False
        ),
    )
    def k(t_hbm, table_hbm, out_hbm, idx_v, table_v, out_v):
        wid = lax.axis_index("s") * nc + lax.axis_index("c")
        base = wid * b_per_w
        pltpu.sync_copy(t_hbm.at[pl.ds(base, b_per_w)], idx_v)
        pltpu.sync_copy(table_hbm, table_v)
        lane = lax.broadcasted_iota(jnp.int32, (nl,), 0)
        lane_d = lane * _D

        def body(chunk, carry):
            t_vec = idx_v[pl.ds(chunk * nl, nl)]
            t_d = t_vec * _D
            obase = chunk * (nl * _D) + lane_d
            for c in range(_D):
                vals = plsc.load_gather(table_v, [t_d + c])
                plsc.store_scatter(out_v, [obase + c], vals)
            return carry

        lax.fori_loop(0, n_chunks, body, 0)
        pltpu.sync_copy(out_v, out_hbm.at[pl.ds(base * _D, b_per_w * _D)])

    _cached["k"] = k
    return k


def kernel(t, table):
    k = _make_kernel()
    out = k(t.astype(jnp.int32), table.astype(jnp.float32).reshape(-1))
    return out.reshape(_B, _D)


# speedup vs baseline: 2.5176x; 1.3789x over previous
import functools

import jax
import jax.numpy as jnp
from jax import lax
from jax.experimental import pallas as pl
from jax.experimental.pallas import tpu as pltpu
from jax.experimental.pallas import tpu_sc as plsc

_B = 16384
_V = 10
_D = 32

_cached = {}


def _make_kernel():
    if "k" in _cached:
        return _cached["k"]
    info = plsc.get_sparse_core_info()
    nc, ns, nl = info.num_cores, info.num_subcores, info.num_lanes
    nw = nc * ns
    b_per_w = _B // nw
    n_chunks = b_per_w // nl
    mesh = plsc.VectorSubcoreMesh(core_axis_name="c", subcore_axis_name="s")

    @functools.partial(
        pl.kernel,
        mesh=mesh,
        out_type=jax.ShapeDtypeStruct((_B * _D,), jnp.float32),
        scratch_types=[
            pltpu.VMEM((b_per_w,), jnp.int32),
            pltpu.VMEM((_V * _D,), jnp.float32),
            pltpu.VMEM((b_per_w * _D,), jnp.float32),
        ],
        compiler_params=pltpu.CompilerParams(
            use_tc_tiling_on_sc=False, needs_layout_passes=False
        ),
    )
    def k(t_hbm, table_hbm, out_hbm, idx_s, table_v, out_v):
        wid = lax.axis_index("s") * nc + lax.axis_index("c")
        base = wid * b_per_w
        pltpu.sync_copy(t_hbm.at[pl.ds(base, b_per_w)], idx_s)
        pltpu.sync_copy(table_hbm, table_v)
        def body(i, carry):
            r0 = i * nl
            t_vec = idx_s[pl.ds(r0, nl)] * _D
            for j in range(nl):
                a = t_vec[j]
                o = (r0 + j) * _D
                out_v[pl.ds(o, nl)] = table_v[pl.ds(a, nl)]
                out_v[pl.ds(o + nl, nl)] = table_v[pl.ds(a + nl, nl)]
            return carry

        lax.fori_loop(0, n_chunks, body, 0)
        pltpu.sync_copy(out_v, out_hbm.at[pl.ds(base * _D, b_per_w * _D)])

    _cached["k"] = k
    return k


def kernel(t, table):
    k = _make_kernel()
    out = k(t.astype(jnp.int32), table.astype(jnp.float32).reshape(-1))
    return out.reshape(_B, _D)


# speedup vs baseline: 2.8637x; 1.1375x over previous
import functools

import jax
import jax.numpy as jnp
from jax import lax
from jax.experimental import pallas as pl
from jax.experimental.pallas import tpu as pltpu
from jax.experimental.pallas import tpu_sc as plsc

_B = 16384
_V = 10
_D = 32

_cached = {}


def _make_kernel():
    if "k" in _cached:
        return _cached["k"]
    info = plsc.get_sparse_core_info()
    nc, ns, nl = info.num_cores, info.num_subcores, info.num_lanes
    nw = nc * ns
    b_per_w = _B // nw
    n_chunks = b_per_w // nl
    mesh = plsc.VectorSubcoreMesh(core_axis_name="c", subcore_axis_name="s")

    @functools.partial(
        pl.kernel,
        mesh=mesh,
        out_type=jax.ShapeDtypeStruct((_B, _D), jnp.float32),
        scratch_types=[
            pltpu.VMEM((b_per_w,), jnp.int32),
            pltpu.VMEM((_V, _D), jnp.float32),
            pltpu.VMEM((b_per_w, _D), jnp.float32),
        ],
        compiler_params=pltpu.CompilerParams(needs_layout_passes=False),
    )
    def k(t_hbm, table_hbm, out_hbm, idx_s, table_v, out_v):
        wid = lax.axis_index("s") * nc + lax.axis_index("c")
        base = wid * b_per_w
        pltpu.sync_copy(t_hbm.at[pl.ds(base, b_per_w)], idx_s)
        pltpu.sync_copy(table_hbm, table_v)

        def body(i, carry):
            r0 = i * nl
            t_vec = idx_s[pl.ds(r0, nl)]
            for j in range(nl):
                a = t_vec[j]
                r = r0 + j
                out_v[r, pl.ds(0, nl)] = table_v[a, pl.ds(0, nl)]
                out_v[r, pl.ds(nl, nl)] = table_v[a, pl.ds(nl, nl)]
            return carry

        lax.fori_loop(0, n_chunks, body, 0)
        pltpu.sync_copy(out_v, out_hbm.at[pl.ds(base, b_per_w)])

    _cached["k"] = k
    return k


def kernel(t, table):
    k = _make_kernel()
    return k(t.astype(jnp.int32), table.astype(jnp.float32))


# speedup vs baseline: 2.9485x; 1.0296x over previous
import functools

import jax
import jax.numpy as jnp
from jax import lax
from jax.experimental import pallas as pl
from jax.experimental.pallas import tpu as pltpu
from jax.experimental.pallas import tpu_sc as plsc

_B = 16384
_V = 10
_D = 32

_cached = {}


def _make_kernel():
    if "k" in _cached:
        return _cached["k"]
    info = plsc.get_sparse_core_info()
    nc, ns, nl = info.num_cores, info.num_subcores, info.num_lanes
    nw = nc * ns
    b_per_w = _B // nw
    n_q = 4
    rows_q = b_per_w // n_q
    mesh = plsc.VectorSubcoreMesh(core_axis_name="c", subcore_axis_name="s")

    @functools.partial(
        pl.kernel,
        mesh=mesh,
        out_type=jax.ShapeDtypeStruct((_B, _D), jnp.float32),
        scratch_types=[
            pltpu.VMEM((b_per_w,), jnp.int32),
            pltpu.VMEM((_V, _D), jnp.float32),
            pltpu.VMEM((b_per_w, _D), jnp.float32),
            pltpu.SemaphoreType.DMA,
            pltpu.SemaphoreType.DMA,
            pltpu.SemaphoreType.DMA,
        ],
        compiler_params=pltpu.CompilerParams(needs_layout_passes=False),
    )
    def k(t_hbm, table_hbm, out_hbm, idx_v, table_v, out_v, sem_i, sem_t, sem_o):
        wid = lax.axis_index("s") * nc + lax.axis_index("c")
        base = wid * b_per_w
        cp_i = pltpu.async_copy(t_hbm.at[pl.ds(base, b_per_w)], idx_v, sem_i)
        cp_t = pltpu.async_copy(table_hbm, table_v, sem_t)
        cp_i.wait()
        cp_t.wait()

        def body(i, carry):
            r0 = i * nl
            t_vec = idx_v[pl.ds(r0, nl)]
            for j in range(nl):
                r = r0 + j
                a = t_vec[j]
                out_v[r, pl.ds(0, nl)] = table_v[a, pl.ds(0, nl)]
                out_v[r, pl.ds(nl, nl)] = table_v[a, pl.ds(nl, nl)]
            return carry

        outcps = []
        for q in range(n_q):
            lax.fori_loop(q * rows_q // nl, (q + 1) * rows_q // nl, body, 0)
            outcps.append(
                pltpu.async_copy(
                    out_v.at[pl.ds(q * rows_q, rows_q)],
                    out_hbm.at[pl.ds(base + q * rows_q, rows_q)],
                    sem_o,
                )
            )
        for cp in outcps:
            cp.wait()

    _cached["k"] = k
    return k


def kernel(t, table):
    k = _make_kernel()
    return k(t.astype(jnp.int32), table.astype(jnp.float32))


# speedup vs baseline: 3.0679x; 1.0405x over previous
import functools

import jax
import jax.numpy as jnp
from jax import lax
from jax.experimental import pallas as pl
from jax.experimental.pallas import tpu as pltpu
from jax.experimental.pallas import tpu_sc as plsc

_B = 16384
_V = 10
_D = 32
_DP = 128

_cached = {}


def _make_kernel():
    if "k" in _cached:
        return _cached["k"]
    info = plsc.get_sparse_core_info()
    nc, ns, nl = info.num_cores, info.num_subcores, info.num_lanes
    nw = nc * ns
    b_per_w = _B // nw
    n_q = 4
    rows_q = b_per_w // n_q
    mesh = plsc.VectorSubcoreMesh(core_axis_name="c", subcore_axis_name="s")

    @functools.partial(
        pl.kernel,
        mesh=mesh,
        out_type=jax.ShapeDtypeStruct((_B, _D), jnp.float32),
        scratch_types=[
            pltpu.VMEM((b_per_w,), jnp.int32),
            pltpu.VMEM((_V, _D), jnp.float32),
            pltpu.VMEM((_V, _DP), jnp.float32),
            pltpu.VMEM_SHARED((_V, _DP), jnp.float32),
            pltpu.VMEM((2, rows_q, _DP), jnp.float32),
            pltpu.VMEM((b_per_w, _D), jnp.float32),
            pltpu.SemaphoreType.DMA,
            pltpu.SemaphoreType.DMA,
            pltpu.SemaphoreType.DMA,
            pltpu.SemaphoreType.DMA,
            pltpu.SemaphoreType.DMA,
        ],
        compiler_params=pltpu.CompilerParams(needs_layout_passes=False),
    )
    def k(t_hbm, table_hbm, out_hbm, idx_v, table_v, tpad_v, tpad_sh, rows_v,
          out_v, sem_i, sem_t, sem_g0, sem_g1, sem_o):
        sid = lax.axis_index("s")
        wid = sid * nc + lax.axis_index("c")
        base = wid * b_per_w
        cp_i = pltpu.async_copy(t_hbm.at[pl.ds(base, b_per_w)], idx_v, sem_i)

        @pl.when(sid == 0)
        def _stage_table():
            pltpu.sync_copy(table_hbm, table_v)
            for v in range(_V):
                tpad_v[v, pl.ds(0, nl)] = table_v[v, pl.ds(0, nl)]
                tpad_v[v, pl.ds(nl, nl)] = table_v[v, pl.ds(nl, nl)]
            pltpu.sync_copy(tpad_v, tpad_sh)

        plsc.subcore_barrier()
        cp_i.wait()

        g_sems = [sem_g0, sem_g1]

        def fire_gather(q):
            return pltpu.async_copy(
                tpad_sh.at[idx_v.at[pl.ds(q * rows_q, rows_q)]],
                rows_v.at[q % 2],
                g_sems[q % 2],
            )

        cps_g = {0: fire_gather(0)}
        outcps = []
        for q in range(n_q):
            if q + 1 < n_q:
                cps_g[q + 1] = fire_gather(q + 1)
            cps_g[q].wait()
            buf = q % 2

            def body(i, carry, buf=buf, q=q):
                r0 = i * nl
                for j in range(nl):
                    r = r0 + j
                    out_v[q * rows_q + r, pl.ds(0, nl)] = rows_v[buf, r, pl.ds(0, nl)]
                    out_v[q * rows_q + r, pl.ds(nl, nl)] = rows_v[buf, r, pl.ds(nl, nl)]
                return carry

            lax.fori_loop(0, rows_q // nl, body, 0)
            outcps.append(
                pltpu.async_copy(
                    out_v.at[pl.ds(q * rows_q, rows_q)],
                    out_hbm.at[pl.ds(base + q * rows_q, rows_q)],
                    sem_o,
                )
            )
        for cp in outcps:
            cp.wait()

    _cached["k"] = k
    return k


def kernel(t, table):
    k = _make_kernel()
    return k(t.astype(jnp.int32), table.astype(jnp.float32))
